# Initial kernel scaffold; baseline (speedup 1.0000x reference)
import jax
import jax.numpy as jnp
from jax.experimental import pallas as pl


def kernel(x, embedding):
    raise NotImplementedError("write your pallas kernel here")


# speedup vs baseline: 3.1827x; 3.1827x over previous
import jax
import jax.numpy as jnp
from jax.experimental import pallas as pl


def _copy_body(emb_ref, out_ref):
    out_ref[...] = emb_ref[...]


def kernel(x, embedding):
    seq_len = x.shape[1]
    d_model = embedding.shape[1]
    block = 1024
    out = pl.pallas_call(
        _copy_body,
        grid=(seq_len // block,),
        in_specs=[pl.BlockSpec((block, d_model), lambda i: (i, 0))],
        out_specs=pl.BlockSpec((block, d_model), lambda i: (i, 0)),
        out_shape=jax.ShapeDtypeStruct((seq_len, d_model), embedding.dtype),
    )(embedding)
    return out[None]
